# baseline (speedup 1.0000x reference)
import functools

import jax
import jax.numpy as jnp
from jax import lax
from jax.experimental import pallas as pl
from jax.experimental.pallas import tpu as pltpu
from jax.experimental.pallas import tpu_sc as plsc

H = 16
P = 4095
D = 64
L = 2048
W = 4096
MT = 31
CPW = 8


HB = 8


def _tc_sum_body(emb_ref, out_ref):
    ones = jnp.ones((1, D), jnp.float32)
    for b in range(HB):
        x = emb_ref[b]
        s = lax.dot_general(ones, x, (((1,), (1,)), ((), ())),
                            preferred_element_type=jnp.float32)
        out_ref[b] = jnp.pad(s, ((0, 0), (0, W - P)))


def _tc_sum(emb):
    return pl.pallas_call(
        _tc_sum_body,
        grid=(H // HB,),
        in_specs=[pl.BlockSpec((HB, P, D), lambda g: (g, 0, 0))],
        out_specs=pl.BlockSpec((HB, 1, W), lambda g: (g, 0, 0)),
        out_shape=jax.ShapeDtypeStruct((H, 1, W), jnp.float32),
        compiler_params=pltpu.CompilerParams(allow_input_fusion=[True]),
    )(emb).reshape(H, W)


@functools.partial(
    pl.kernel,
    mesh=plsc.VectorSubcoreMesh(core_axis_name="c", subcore_axis_name="s"),
    out_type=jax.ShapeDtypeStruct((H, L // 8, L // 128, 8, 128), jnp.float32),
    scratch_types=[
        pltpu.VMEM((W,), jnp.float32),
        pltpu.VMEM((2, MT, 8, 128), jnp.float32),
        pltpu.SemaphoreType.DMA,
    ],
    compiler_params=pltpu.CompilerParams(use_tc_tiling_on_sc=False),
)
def _sc_expand(s_hbm, out_hbm, s_v, u_v, sem):
    wid = lax.axis_index("s") * 2 + lax.axis_index("c")
    h = wid // 2
    half = wid % 2
    pltpu.sync_copy(s_hbm.at[h], s_v)
    c0 = half * CPW

    def build(cc, buf):
        base = (c0 + cc) * 8 + 1

        def body(mt, carry):
            for ri in range(8):
                for w in range(8):
                    off = 128 * mt + 16 * w - base - ri
                    u_v[buf, mt - 1, ri, pl.ds(16 * w, 16)] = s_v[pl.ds(off, 16)]
            return carry

        lax.fori_loop(1, MT + 1, body, 0)

    def fire(cc, buf):
        c = c0 + cc
        cps = []
        for u in range(16):
            cps.append(pltpu.async_copy(
                u_v.at[buf, pl.ds(15 - u, 16)],
                out_hbm.at[h, 16 * u + c],
                sem,
            ))
        return cps

    build(0, 0)
    pending = [fire(0, 0)]
    build(1, 1)
    pending.append(fire(1, 1))
    for cc in range(2, CPW):
        for cp in pending.pop(0):
            cp.wait()
        build(cc, cc % 2)
        pending.append(fire(cc, cc % 2))
    for cps in pending:
        for cp in cps:
            cp.wait()


def kernel(length, embeddings):
    s = _tc_sum(embeddings[..., 0])
    out5 = _sc_expand(s)
    return out5.transpose(0, 1, 3, 2, 4).reshape(H, L, L)


# speedup vs baseline: 1.0060x; 1.0060x over previous
import functools

import jax
import jax.numpy as jnp
from jax import lax
from jax.experimental import pallas as pl
from jax.experimental.pallas import tpu as pltpu
from jax.experimental.pallas import tpu_sc as plsc

H = 16
P = 4095
D = 64
L = 2048
W = 4096
MT = 31
CPW = 8


HB = 4


def _tc_sum_body(emb_ref, out_ref):
    ones = jnp.ones((1, D), jnp.float32)
    for b in range(HB):
        x = emb_ref[b]
        s = lax.dot_general(ones, x, (((1,), (1,)), ((), ())),
                            preferred_element_type=jnp.float32)
        out_ref[b] = jnp.pad(s, ((0, 0), (0, W - P)))


def _tc_sum(emb):
    return pl.pallas_call(
        _tc_sum_body,
        grid=(H // HB,),
        in_specs=[pl.BlockSpec((HB, P, D), lambda g: (g, 0, 0))],
        out_specs=pl.BlockSpec((HB, 1, W), lambda g: (g, 0, 0)),
        out_shape=jax.ShapeDtypeStruct((H, 1, W), jnp.float32),
        compiler_params=pltpu.CompilerParams(allow_input_fusion=[True]),
    )(emb).reshape(H, W)


@functools.partial(
    pl.kernel,
    mesh=plsc.VectorSubcoreMesh(core_axis_name="c", subcore_axis_name="s"),
    out_type=jax.ShapeDtypeStruct((H, L // 8, L // 128, 8, 128), jnp.float32),
    scratch_types=[
        pltpu.VMEM((W,), jnp.float32),
        pltpu.VMEM((2, MT, 8, 128), jnp.float32),
        pltpu.SemaphoreType.DMA,
    ],
    compiler_params=pltpu.CompilerParams(use_tc_tiling_on_sc=False),
)
def _sc_expand(s_hbm, out_hbm, s_v, u_v, sem):
    wid = lax.axis_index("s") * 2 + lax.axis_index("c")
    h = wid // 2
    half = wid % 2
    pltpu.sync_copy(s_hbm.at[h], s_v)
    c0 = half * CPW

    def build(cc, buf):
        base = (c0 + cc) * 8 + 1

        def body(mt, carry):
            for ri in range(8):
                for w in range(8):
                    off = 128 * mt + 16 * w - base - ri
                    u_v[buf, mt - 1, ri, pl.ds(16 * w, 16)] = s_v[pl.ds(off, 16)]
            return carry

        lax.fori_loop(1, MT + 1, body, 0)

    def fire(cc, buf):
        c = c0 + cc
        cps = []
        for u in range(16):
            cps.append(pltpu.async_copy(
                u_v.at[buf, pl.ds(15 - u, 16)],
                out_hbm.at[h, 16 * u + c],
                sem,
            ))
        return cps

    build(0, 0)
    pending = [fire(0, 0)]
    build(1, 1)
    pending.append(fire(1, 1))
    for cc in range(2, CPW):
        for cp in pending.pop(0):
            cp.wait()
        build(cc, cc % 2)
        pending.append(fire(cc, cc % 2))
    for cps in pending:
        for cp in cps:
            cp.wait()


def kernel(length, embeddings):
    s = _tc_sum(embeddings[..., 0])
    out5 = _sc_expand(s)
    return out5.transpose(0, 1, 3, 2, 4).reshape(H, L, L)
